# baseline (speedup 1.0000x reference)
import functools

import numpy as np
import jax
import jax.numpy as jnp
from jax import lax
from jax.experimental import pallas as pl
from jax.experimental.pallas import tpu as pltpu
from jax.experimental.pallas import tpu_sc as plsc

B = 4096
N_CAT = 26
N_CONT = 13
VOCAB = 100000
EMB = 64
TOTAL = N_CONT + N_CAT * EMB
HIDDEN = 512
NOISE_P = 0.1

NC, NS = 2, 16
NW = NC * NS
D_E = N_CAT * EMB
VPW = D_E // NW
LANES = 16

BM = 256


_noise_cache = {}


def _rotl32(x, r):
    return ((x << np.uint32(r)) | (x >> np.uint32(32 - r))).astype(np.uint32)


def _threefry2x32(k1, k2, x0, x1):
    x0 = x0.astype(np.uint32).copy()
    x1 = x1.astype(np.uint32).copy()
    ks0, ks1 = np.uint32(k1), np.uint32(k2)
    ks = [ks0, ks1, np.uint32(ks0 ^ ks1 ^ np.uint32(0x1BD11BDA))]
    rotations = [(13, 15, 26, 6), (17, 29, 16, 24)]
    x0 = (x0 + ks0).astype(np.uint32)
    x1 = (x1 + ks1).astype(np.uint32)
    for i in range(5):
        for r in rotations[i % 2]:
            x0 = (x0 + x1).astype(np.uint32)
            x1 = (_rotl32(x1, r) ^ x0).astype(np.uint32)
        x0 = (x0 + ks[(i + 1) % 3]).astype(np.uint32)
        x1 = (x1 + ks[(i + 2) % 3] + np.uint32(i + 1)).astype(np.uint32)
    return x0, x1


def _np_random_bits(key, shape):
    n = int(np.prod(shape))
    idx = np.arange(n, dtype=np.uint64)
    b1, b2 = _threefry2x32(key[0], key[1],
                           (idx >> np.uint64(32)).astype(np.uint32),
                           (idx & np.uint64(0xFFFFFFFF)).astype(np.uint32))
    return (b1 ^ b2).reshape(shape)


def _np_split(key, num=2):
    b1, b2 = _threefry2x32(key[0], key[1], np.zeros(num, np.uint32),
                           np.arange(num, dtype=np.uint32))
    return list(zip(b1, b2))


def _noise_constants():
    if not _noise_cache:
        kmask, kperm = _np_split((np.uint32(0), np.uint32(42)))
        bits = _np_random_bits(kmask, (B, TOTAL))
        u = ((bits >> np.uint32(9)) | np.uint32(0x3F800000)).view(np.float32)
        u = np.maximum(np.float32(0.0), u - np.float32(1.0))
        mask_np = u < np.float32(NOISE_P)
        perm = np.arange(B, dtype=np.int32)
        cur = kperm
        for _ in range(2):
            cur, sub = _np_split(cur)
            sort_keys = _np_random_bits(sub, (B,))
            assert len(np.unique(sort_keys)) == B
            perm = perm[np.argsort(sort_keys, kind="stable")]
        _noise_cache["mask_f"] = mask_np.astype(np.float32)
        _noise_cache["mask_eT_u8"] = np.ascontiguousarray(
            mask_np[:, N_CONT:].T).astype(np.uint8)
        _noise_cache["mask_c_f"] = np.ascontiguousarray(
            mask_np[:, :N_CONT]).astype(np.float32)
        _noise_cache["perm"] = perm
    return _noise_cache


H0 = 50048
H1 = VOCAB - H0


def _sc_sweep_body(table_t, idx0, idx1, out0, out1, vha, vhb,
                   l0a, l0b, l1a, l1b, tmp, o0v, o1v, s0, s1):
    cid = lax.axis_index("c")
    sid = lax.axis_index("s")
    wid = sid * NC + cid
    base = wid * VPW
    last = base + VPW - 1
    f_first = base // EMB
    f_last = last // EMB

    pltpu.async_copy(table_t.at[base].at[pl.ds(0, H0)], vha, s0)

    def partition(idxref, feat, la, lb):
        pltpu.sync_copy(idxref.at[feat], tmp)

        def pc(k, carry):
            na, nb = carry
            ids = tmp[pl.ds(k * LANES, LANES)]
            pos = lax.iota(jnp.int32, LANES) + k * LANES
            packed = jnp.bitwise_or(jnp.left_shift(pos, 17), ids)
            m = ids < H0
            plsc.store_compressed(la.at[pl.ds(na, LANES)], packed, mask=m)
            plsc.store_compressed(lb.at[pl.ds(nb, LANES)], packed, mask=~m)
            inc = jnp.max(plsc.all_reduce_population_count(m))
            return na + inc, nb + (LANES - inc)

        return lax.fori_loop(0, B // LANES, pc, (0, 0), unroll=4)

    def gather_list(lref, n, vh, sub, ov):
        def gc(k, carry):
            o = k * LANES
            p = lref[pl.ds(o, LANES)]
            m = (lax.iota(jnp.int32, LANES) + o) < n
            ids = jnp.bitwise_and(p, 0x1FFFF) - sub
            pos = lax.shift_right_logical(p, 17)
            g = plsc.load_gather(vh, [ids], mask=m)
            plsc.store_scatter(ov, [pos], g, mask=m)
            return carry

        lax.fori_loop(0, (n + LANES - 1) // LANES, gc, 0)

    def feature(f, carry):
        n0a, n0b = partition(idx0, f, l0a, l0b)
        n1a, n1b = partition(idx1, f, l1a, l1b)
        lo_c = jnp.maximum(base, f * EMB)
        hi_c = jnp.minimum(base + VPW, (f + 1) * EMB)

        def sweep(c, carry2):
            pltpu.make_async_copy(
                table_t.at[c].at[pl.ds(0, H0)], vha, s0).wait()
            pltpu.async_copy(table_t.at[c].at[pl.ds(H0, H1)], vhb, s1)
            gather_list(l0a, n0a, vha, 0, o0v)
            gather_list(l1a, n1a, vha, 0, o1v)
            pltpu.make_async_copy(
                table_t.at[c].at[pl.ds(H0, H1)], vhb, s1).wait()
            cn = jnp.minimum(c + 1, last)
            pltpu.async_copy(table_t.at[cn].at[pl.ds(0, H0)], vha, s0)
            gather_list(l0b, n0b, vhb, H0, o0v)
            gather_list(l1b, n1b, vhb, H0, o1v)
            pltpu.sync_copy(o0v, out0.at[c])
            pltpu.sync_copy(o1v, out1.at[c])
            return carry2

        lax.fori_loop(lo_c, hi_c, sweep, 0)
        return carry

    lax.fori_loop(f_first, f_last + 1, feature, 0)
    pltpu.make_async_copy(table_t.at[last].at[pl.ds(0, H0)], vha, s0).wait()


@functools.cache
def _make_sc_sweep():
    return pl.kernel(
        _sc_sweep_body,
        out_type=(
            jax.ShapeDtypeStruct((D_E, B), jnp.float32),
            jax.ShapeDtypeStruct((D_E, B), jnp.float32),
        ),
        mesh=plsc.VectorSubcoreMesh(core_axis_name="c", subcore_axis_name="s",
                                    num_cores=NC, num_subcores=NS),
        scratch_types=[
            pltpu.VMEM((H0,), jnp.float32),
            pltpu.VMEM((H1,), jnp.float32),
            pltpu.VMEM((B + LANES,), jnp.int32),
            pltpu.VMEM((B + LANES,), jnp.int32),
            pltpu.VMEM((B + LANES,), jnp.int32),
            pltpu.VMEM((B + LANES,), jnp.int32),
            pltpu.VMEM((B,), jnp.int32),
            pltpu.VMEM((B,), jnp.float32),
            pltpu.VMEM((B,), jnp.float32),
            pltpu.SemaphoreType.DMA,
            pltpu.SemaphoreType.DMA,
        ],
        compiler_params=pltpu.CompilerParams(use_tc_tiling_on_sc=True,
                                             needs_layout_passes=False),
    )


def _tc_body(e0t, e1t, met, cont, cperm, mc, w_e, w_c, bias, z):
    x_et = jnp.where(met[...] != 0, e1t[...], e0t[...])
    x_c = jnp.where(mc[...] != 0.0, cperm[...], cont[...])
    acc = jax.lax.dot_general(
        x_et, w_e[...], (((0,), (0,)), ((), ())),
        preferred_element_type=jnp.float32)
    acc = acc + jnp.dot(x_c, w_c[...], preferred_element_type=jnp.float32)
    z[...] = jnp.maximum(acc + bias[...], 0.0)


def _tc_forward(e0t, e1t, met, cont, cperm, mc, w_e, w_c, bias):
    return pl.pallas_call(
        _tc_body,
        grid=(B // BM,),
        in_specs=[
            pl.BlockSpec((D_E, BM), lambda m: (0, m)),
            pl.BlockSpec((D_E, BM), lambda m: (0, m)),
            pl.BlockSpec((D_E, BM), lambda m: (0, m)),
            pl.BlockSpec((BM, N_CONT), lambda m: (m, 0)),
            pl.BlockSpec((BM, N_CONT), lambda m: (m, 0)),
            pl.BlockSpec((BM, N_CONT), lambda m: (m, 0)),
            pl.BlockSpec((D_E, HIDDEN), lambda m: (0, 0)),
            pl.BlockSpec((N_CONT, HIDDEN), lambda m: (0, 0)),
            pl.BlockSpec((1, HIDDEN), lambda m: (0, 0)),
        ],
        out_specs=pl.BlockSpec((BM, HIDDEN), lambda m: (m, 0)),
        out_shape=jax.ShapeDtypeStruct((B, HIDDEN), jnp.float32),
    )(e0t, e1t, met, cont, cperm, mc, w_e, w_c, bias)


def kernel(continuous, categorical, emb_tables, W, b):
    nz = _noise_constants()
    perm = nz["perm"]

    cat = categorical.astype(jnp.int32)
    idx0 = cat.T
    idx1 = cat[perm, :].T

    table_t = emb_tables.transpose(0, 2, 1).reshape(D_E, VOCAB)
    e0t, e1t = _make_sc_sweep()(table_t, idx0, idx1)

    z = _tc_forward(
        e0t, e1t, jnp.asarray(nz["mask_eT_u8"]),
        continuous, continuous[perm, :], jnp.asarray(nz["mask_c_f"]),
        W[N_CONT:, :], W[:N_CONT, :], b.reshape(1, HIDDEN),
    )
    return z, jnp.asarray(nz["mask_f"])


# speedup vs baseline: 1.0155x; 1.0155x over previous
import functools

import numpy as np
import jax
import jax.numpy as jnp
from jax import lax
from jax.experimental import pallas as pl
from jax.experimental.pallas import tpu as pltpu
from jax.experimental.pallas import tpu_sc as plsc

B = 4096
N_CAT = 26
N_CONT = 13
VOCAB = 100000
EMB = 64
TOTAL = N_CONT + N_CAT * EMB
HIDDEN = 512
NOISE_P = 0.1

NC, NS = 2, 16
NW = NC * NS
D_E = N_CAT * EMB
VPW = D_E // NW
LANES = 16

BM = 256


_noise_cache = {}


def _rotl32(x, r):
    return ((x << np.uint32(r)) | (x >> np.uint32(32 - r))).astype(np.uint32)


def _threefry2x32(k1, k2, x0, x1):
    x0 = x0.astype(np.uint32).copy()
    x1 = x1.astype(np.uint32).copy()
    ks0, ks1 = np.uint32(k1), np.uint32(k2)
    ks = [ks0, ks1, np.uint32(ks0 ^ ks1 ^ np.uint32(0x1BD11BDA))]
    rotations = [(13, 15, 26, 6), (17, 29, 16, 24)]
    x0 = (x0 + ks0).astype(np.uint32)
    x1 = (x1 + ks1).astype(np.uint32)
    for i in range(5):
        for r in rotations[i % 2]:
            x0 = (x0 + x1).astype(np.uint32)
            x1 = (_rotl32(x1, r) ^ x0).astype(np.uint32)
        x0 = (x0 + ks[(i + 1) % 3]).astype(np.uint32)
        x1 = (x1 + ks[(i + 2) % 3] + np.uint32(i + 1)).astype(np.uint32)
    return x0, x1


def _np_random_bits(key, shape):
    n = int(np.prod(shape))
    idx = np.arange(n, dtype=np.uint64)
    b1, b2 = _threefry2x32(key[0], key[1],
                           (idx >> np.uint64(32)).astype(np.uint32),
                           (idx & np.uint64(0xFFFFFFFF)).astype(np.uint32))
    return (b1 ^ b2).reshape(shape)


def _np_split(key, num=2):
    b1, b2 = _threefry2x32(key[0], key[1], np.zeros(num, np.uint32),
                           np.arange(num, dtype=np.uint32))
    return list(zip(b1, b2))


def _noise_constants():
    if not _noise_cache:
        kmask, kperm = _np_split((np.uint32(0), np.uint32(42)))
        bits = _np_random_bits(kmask, (B, TOTAL))
        u = ((bits >> np.uint32(9)) | np.uint32(0x3F800000)).view(np.float32)
        u = np.maximum(np.float32(0.0), u - np.float32(1.0))
        mask_np = u < np.float32(NOISE_P)
        perm = np.arange(B, dtype=np.int32)
        cur = kperm
        for _ in range(2):
            cur, sub = _np_split(cur)
            sort_keys = _np_random_bits(sub, (B,))
            assert len(np.unique(sort_keys)) == B
            perm = perm[np.argsort(sort_keys, kind="stable")]
        _noise_cache["mask_f"] = mask_np.astype(np.float32)
        _noise_cache["mask_eT_u8"] = np.ascontiguousarray(
            mask_np[:, N_CONT:].T).astype(np.uint8)
        _noise_cache["mask_c_f"] = np.ascontiguousarray(
            mask_np[:, :N_CONT]).astype(np.float32)
        _noise_cache["perm"] = perm
    return _noise_cache


H0 = 50048
H1 = VOCAB - H0


def _sc_sweep_body(table_t, idx0, idx1, out0, out1, vha, vhb,
                   l0a, l0b, l1a, l1b, tmp, o0v, o1v, s0, s1, osem):
    cid = lax.axis_index("c")
    sid = lax.axis_index("s")
    wid = sid * NC + cid
    base = wid * VPW
    last = base + VPW - 1
    f_first = base // EMB
    f_last = last // EMB

    pltpu.async_copy(table_t.at[base].at[pl.ds(0, H0)], vha, s0)

    def partition(idxref, feat, la, lb):
        pltpu.sync_copy(idxref.at[feat], tmp)

        def pc(k, carry):
            na, nb = carry
            ids = tmp[pl.ds(k * LANES, LANES)]
            pos = lax.iota(jnp.int32, LANES) + k * LANES
            packed = jnp.bitwise_or(jnp.left_shift(pos, 17), ids)
            m = ids < H0
            plsc.store_compressed(la.at[pl.ds(na, LANES)], packed, mask=m)
            plsc.store_compressed(lb.at[pl.ds(nb, LANES)], packed, mask=~m)
            inc = jnp.max(plsc.all_reduce_population_count(m))
            return na + inc, nb + (LANES - inc)

        return lax.fori_loop(0, B // LANES, pc, (0, 0), unroll=4)

    def gather_list(lref, n, vh, sub, ov):
        def gc(k, carry):
            o = k * LANES
            p = lref[pl.ds(o, LANES)]
            m = (lax.iota(jnp.int32, LANES) + o) < n
            ids = jnp.bitwise_and(p, 0x1FFFF) - sub
            pos = lax.shift_right_logical(p, 17)
            g = plsc.load_gather(vh, [ids], mask=m)
            plsc.store_scatter(ov, [pos], g, mask=m)
            return carry

        lax.fori_loop(0, (n + LANES - 1) // LANES, gc, 0)

    def feature(f, carry):
        n0a, n0b = partition(idx0, f, l0a, l0b)
        n1a, n1b = partition(idx1, f, l1a, l1b)
        lo_c = jnp.maximum(base, f * EMB)
        hi_c = jnp.minimum(base + VPW, (f + 1) * EMB)

        def sweep(c, carry2):
            pltpu.make_async_copy(
                table_t.at[c].at[pl.ds(0, H0)], vha, s0).wait()
            pltpu.async_copy(table_t.at[c].at[pl.ds(H0, H1)], vhb, s1)

            @pl.when(c > base)
            def _drain_outs():
                pltpu.make_async_copy(out0.at[c], o0v, osem).wait()
                pltpu.make_async_copy(out1.at[c], o1v, osem).wait()

            gather_list(l0a, n0a, vha, 0, o0v)
            gather_list(l1a, n1a, vha, 0, o1v)
            pltpu.make_async_copy(
                table_t.at[c].at[pl.ds(H0, H1)], vhb, s1).wait()
            cn = jnp.minimum(c + 1, last)
            pltpu.async_copy(table_t.at[cn].at[pl.ds(0, H0)], vha, s0)
            gather_list(l0b, n0b, vhb, H0, o0v)
            gather_list(l1b, n1b, vhb, H0, o1v)
            pltpu.async_copy(o0v, out0.at[c], osem)
            pltpu.async_copy(o1v, out1.at[c], osem)
            return carry2

        lax.fori_loop(lo_c, hi_c, sweep, 0)
        return carry

    lax.fori_loop(f_first, f_last + 1, feature, 0)
    pltpu.make_async_copy(table_t.at[last].at[pl.ds(0, H0)], vha, s0).wait()
    pltpu.make_async_copy(out0.at[last], o0v, osem).wait()
    pltpu.make_async_copy(out1.at[last], o1v, osem).wait()


@functools.cache
def _make_sc_sweep():
    return pl.kernel(
        _sc_sweep_body,
        out_type=(
            jax.ShapeDtypeStruct((D_E, B), jnp.float32),
            jax.ShapeDtypeStruct((D_E, B), jnp.float32),
        ),
        mesh=plsc.VectorSubcoreMesh(core_axis_name="c", subcore_axis_name="s",
                                    num_cores=NC, num_subcores=NS),
        scratch_types=[
            pltpu.VMEM((H0,), jnp.float32),
            pltpu.VMEM((H1,), jnp.float32),
            pltpu.VMEM((B + LANES,), jnp.int32),
            pltpu.VMEM((B + LANES,), jnp.int32),
            pltpu.VMEM((B + LANES,), jnp.int32),
            pltpu.VMEM((B + LANES,), jnp.int32),
            pltpu.VMEM((B,), jnp.int32),
            pltpu.VMEM((B,), jnp.float32),
            pltpu.VMEM((B,), jnp.float32),
            pltpu.SemaphoreType.DMA,
            pltpu.SemaphoreType.DMA,
            pltpu.SemaphoreType.DMA,
        ],
        compiler_params=pltpu.CompilerParams(use_tc_tiling_on_sc=True,
                                             needs_layout_passes=False),
    )


def _tc_body(e0t, e1t, met, cont, cperm, mc, w_e, w_c, bias, z):
    x_et = jnp.where(met[...] != 0, e1t[...], e0t[...])
    x_c = jnp.where(mc[...] != 0.0, cperm[...], cont[...])
    acc = jax.lax.dot_general(
        x_et, w_e[...], (((0,), (0,)), ((), ())),
        preferred_element_type=jnp.float32)
    acc = acc + jnp.dot(x_c, w_c[...], preferred_element_type=jnp.float32)
    z[...] = jnp.maximum(acc + bias[...], 0.0)


def _tc_forward(e0t, e1t, met, cont, cperm, mc, w_e, w_c, bias):
    return pl.pallas_call(
        _tc_body,
        grid=(B // BM,),
        in_specs=[
            pl.BlockSpec((D_E, BM), lambda m: (0, m)),
            pl.BlockSpec((D_E, BM), lambda m: (0, m)),
            pl.BlockSpec((D_E, BM), lambda m: (0, m)),
            pl.BlockSpec((BM, N_CONT), lambda m: (m, 0)),
            pl.BlockSpec((BM, N_CONT), lambda m: (m, 0)),
            pl.BlockSpec((BM, N_CONT), lambda m: (m, 0)),
            pl.BlockSpec((D_E, HIDDEN), lambda m: (0, 0)),
            pl.BlockSpec((N_CONT, HIDDEN), lambda m: (0, 0)),
            pl.BlockSpec((1, HIDDEN), lambda m: (0, 0)),
        ],
        out_specs=pl.BlockSpec((BM, HIDDEN), lambda m: (m, 0)),
        out_shape=jax.ShapeDtypeStruct((B, HIDDEN), jnp.float32),
    )(e0t, e1t, met, cont, cperm, mc, w_e, w_c, bias)


def kernel(continuous, categorical, emb_tables, W, b):
    nz = _noise_constants()
    perm = nz["perm"]

    cat = categorical.astype(jnp.int32)
    idx0 = cat.T
    idx1 = cat[perm, :].T

    table_t = emb_tables.transpose(0, 2, 1).reshape(D_E, VOCAB)
    e0t, e1t = _make_sc_sweep()(table_t, idx0, idx1)

    z = _tc_forward(
        e0t, e1t, jnp.asarray(nz["mask_eT_u8"]),
        continuous, continuous[perm, :], jnp.asarray(nz["mask_c_f"]),
        W[N_CONT:, :], W[:N_CONT, :], b.reshape(1, HIDDEN),
    )
    return z, jnp.asarray(nz["mask_f"])
